# baseline (speedup 1.0000x reference)
import functools

import jax
import jax.numpy as jnp
from jax import lax
from jax.experimental import pallas as pl
from jax.experimental.pallas import tpu as pltpu
from jax.experimental.pallas import tpu_sc as plsc


def _moe_body(x_ref, wg_ref, we_ref, be_ref, y_ref, logit_ref):
    x = x_ref[...]
    y_ref[...] = (jnp.dot(x, we_ref[...], preferred_element_type=jnp.float32)
                  + be_ref[...])
    logit_ref[...] = jnp.dot(x, wg_ref[...],
                             preferred_element_type=jnp.float32)


def _sc_stats_body(logits_hbm, imp_hbm, load_hbm, buf, impv, loadv,
                   *, rows_per, num_cores):
    c = lax.axis_index("c")
    s_idx = lax.axis_index("s")
    wid = s_idx * num_cores + c
    base = wid * rows_per * 16
    pltpu.sync_copy(logits_hbm.at[pl.ds(base, rows_per * 16)], buf)
    lane = lax.iota(jnp.int32, 16)

    zeros_i = jnp.zeros((16,), jnp.int32)
    ones_i = jnp.ones((16,), jnp.int32)
    dnums = lax.GatherDimensionNumbers(
        offset_dims=(), collapsed_slice_dims=(0,), start_index_map=(0,))

    def bcast(vec, idx):
        return lax.gather(vec, idx[:, None], dnums, slice_sizes=(1,),
                          mode=lax.GatherScatterMode.PROMISE_IN_BOUNDS)

    def body(i, carry):
        imp, load = carry
        row = buf[pl.ds(i * 16, 16)]
        skey, sval = plsc.sort_key_val(row, lane, descending=True)
        m1v = bcast(skey, zeros_i)
        m2v = bcast(skey, ones_i)
        idx1v = bcast(sval, zeros_i)
        idx2v = bcast(sval, ones_i)
        qv = jnp.exp(m2v - m1v)
        sv = 1.0 + qv
        g1v = 1.0 / sv
        g2v = qv / sv
        one1 = (lane == idx1v).astype(jnp.float32)
        one2 = (lane == idx2v).astype(jnp.float32)
        imp = imp + one1 * g1v + one2 * g2v
        load = load + one1 + one2 * (g2v > 0).astype(jnp.float32)
        return imp, load

    z = jnp.zeros((16,), jnp.float32)
    imp, load = lax.fori_loop(0, rows_per, body, (z, z))
    impv[...] = imp
    loadv[...] = load
    pltpu.sync_copy(impv, imp_hbm.at[wid])
    pltpu.sync_copy(loadv, load_hbm.at[wid])


def _loss_body(imp_ref, load_ref, loss_ref, *, e):
    def cv2(v):
        mean = jnp.sum(v) / e
        var = jnp.sum((v - mean) ** 2) / (e - 1)
        return var / (mean * mean + 1e-10)

    imp = jnp.sum(imp_ref[...], axis=0, keepdims=True)
    load = jnp.sum(load_ref[...], axis=0, keepdims=True)
    loss = cv2(imp) + cv2(load)
    loss_ref[...] = jnp.full((1, 1), loss, dtype=jnp.float32)


def kernel(x, w_gate, w_noise, W_e, b_e):
    del w_noise
    n, d = x.shape
    e = w_gate.shape[1]
    tn = 2048 if n % 2048 == 0 else n
    n_steps = n // tn

    y, logits = pl.pallas_call(
        _moe_body,
        grid=(n_steps,),
        in_specs=[
            pl.BlockSpec((tn, d), lambda i: (i, 0)),
            pl.BlockSpec((d, e), lambda i: (0, 0)),
            pl.BlockSpec((d, d), lambda i: (0, 0)),
            pl.BlockSpec((1, d), lambda i: (0, 0)),
        ],
        out_specs=[
            pl.BlockSpec((tn, d), lambda i: (i, 0)),
            pl.BlockSpec((tn, e), lambda i: (i, 0)),
        ],
        out_shape=[
            jax.ShapeDtypeStruct((n, d), jnp.float32),
            jax.ShapeDtypeStruct((n, e), jnp.float32),
        ],
        compiler_params=pltpu.CompilerParams(
            dimension_semantics=("arbitrary",)),
    )(x, w_gate, W_e, b_e.reshape(1, d))

    info = plsc.get_sparse_core_info()
    nw = info.num_cores * info.num_subcores
    rows_per = n // nw
    mesh = plsc.VectorSubcoreMesh(core_axis_name="c", subcore_axis_name="s")
    imp_parts, load_parts = pl.kernel(
        functools.partial(_sc_stats_body, rows_per=rows_per,
                          num_cores=info.num_cores),
        out_type=[
            jax.ShapeDtypeStruct((nw, 16), jnp.float32),
            jax.ShapeDtypeStruct((nw, 16), jnp.float32),
        ],
        mesh=mesh,
        scratch_types=[
            pltpu.VMEM((rows_per * 16,), jnp.float32),
            pltpu.VMEM((16,), jnp.float32),
            pltpu.VMEM((16,), jnp.float32),
        ],
        compiler_params=pltpu.CompilerParams(needs_layout_passes=False),
    )(logits.reshape(n * e))

    loss = pl.pallas_call(
        functools.partial(_loss_body, e=e),
        out_shape=jax.ShapeDtypeStruct((1, 1), jnp.float32),
    )(imp_parts, load_parts)
    return y, loss[0, 0]


# speedup vs baseline: 1.5398x; 1.5398x over previous
import functools

import jax
import jax.numpy as jnp
from jax.experimental import pallas as pl
from jax.experimental.pallas import tpu as pltpu


def _moe_body(x_ref, wgt_ref, we_ref, be_ref, y_ref, stats_ref, loss_ref,
              *, n_steps):
    e = wgt_ref.shape[0]
    x = x_ref[...]

    y_ref[...] = (jnp.dot(x, we_ref[...], preferred_element_type=jnp.float32)
                  + be_ref[...])

    logits_t = jax.lax.dot_general(
        wgt_ref[...], x, (((1,), (1,)), ((), ())),
        preferred_element_type=jnp.float32)

    row = jax.lax.broadcasted_iota(jnp.int32, logits_t.shape, 0)
    m1 = jnp.max(logits_t, axis=0, keepdims=True)
    idx1 = jnp.min(jnp.where(logits_t == m1, row, e), axis=0, keepdims=True)
    masked = jnp.where(row == idx1, -jnp.inf, logits_t)
    m2 = jnp.max(masked, axis=0, keepdims=True)
    idx2 = jnp.min(jnp.where(masked == m2, row, e), axis=0, keepdims=True)

    q = jnp.exp(m2 - m1)
    s = 1.0 + q
    g1 = 1.0 / s
    g2 = q / s

    one1 = (row == idx1).astype(jnp.float32)
    one2 = (row == idx2).astype(jnp.float32)
    imp_part = jnp.sum(one1 * g1 + one2 * g2, axis=1, keepdims=True)
    load_part = jnp.sum(one1 + one2 * (g2 > 0).astype(jnp.float32),
                        axis=1, keepdims=True)

    @pl.when(pl.program_id(0) == 0)
    def _init():
        stats_ref[...] = jnp.zeros_like(stats_ref)

    stats_ref[:, 0:1] += imp_part
    stats_ref[:, 1:2] += load_part

    @pl.when(pl.program_id(0) == n_steps - 1)
    def _finalize():
        def cv2(v):
            mean = jnp.sum(v) / e
            var = jnp.sum((v - mean) ** 2) / (e - 1)
            return var / (mean * mean + 1e-10)

        loss = cv2(stats_ref[:, 0:1]) + cv2(stats_ref[:, 1:2])
        loss_ref[...] = jnp.full((1, 1), loss, dtype=jnp.float32)


def kernel(x, w_gate, w_noise, W_e, b_e):
    del w_noise
    n, d = x.shape
    e = w_gate.shape[1]
    tn = 2048 if n % 2048 == 0 else n
    n_steps = n // tn

    y, _, loss = pl.pallas_call(
        functools.partial(_moe_body, n_steps=n_steps),
        grid=(n_steps,),
        in_specs=[
            pl.BlockSpec((tn, d), lambda i: (i, 0)),
            pl.BlockSpec((e, d), lambda i: (0, 0)),
            pl.BlockSpec((d, d), lambda i: (0, 0)),
            pl.BlockSpec((1, d), lambda i: (0, 0)),
        ],
        out_specs=[
            pl.BlockSpec((tn, d), lambda i: (i, 0)),
            pl.BlockSpec((e, 2), lambda i: (0, 0)),
            pl.BlockSpec((1, 1), lambda i: (0, 0)),
        ],
        out_shape=[
            jax.ShapeDtypeStruct((n, d), jnp.float32),
            jax.ShapeDtypeStruct((e, 2), jnp.float32),
            jax.ShapeDtypeStruct((1, 1), jnp.float32),
        ],
        compiler_params=pltpu.CompilerParams(
            dimension_semantics=("arbitrary",)),
    )(x, w_gate.T, W_e, b_e.reshape(1, d))

    return y, loss[0, 0]
